# baseline (speedup 1.0000x reference)
import functools

import jax
import jax.numpy as jnp
from jax import lax
from jax.experimental import pallas as pl
from jax.experimental.pallas import tpu as pltpu
from jax.experimental.pallas import tpu_sc as plsc

VOCAB = 1000000
EMBED_DIM = 64
BATCH = 4096
HIST = 200

NUM_CORES = 2
NUM_SUBCORES = 16
NUM_WORKERS = NUM_CORES * NUM_SUBCORES

XROWS = BATCH // NUM_WORKERS
PAIR = 1
NUM_CHUNKS = XROWS // PAIR
NBUF = 4
DEPTH = NBUF - 1


def _embed_kernel(x_hbm, table_hbm, out_hbm, idx_all, rows_v, gsems, ssems):
    wid = lax.axis_index("s") * NUM_CORES + lax.axis_index("c")
    wrow = wid * XROWS

    pltpu.sync_copy(x_hbm.at[pl.ds(wrow, XROWS)], idx_all)

    def gather_copy(c, b):
        src = table_hbm.at[idx_all.at[c]]
        return pltpu.make_async_copy(src, rows_v.at[b], gsems[b])

    def store_copy(c, b):
        dst = out_hbm.at[wrow + c]
        return pltpu.make_async_copy(rows_v.at[b], dst, ssems[b])

    def visit(c, b):
        gather_copy(c, b).wait()
        store_copy(c, b).start()
        h = c + DEPTH
        hb = (b + DEPTH) % NBUF

        @pl.when(h < NUM_CHUNKS)
        def _():
            @pl.when(h >= NBUF)
            def _():
                store_copy(h - NBUF, hb).wait()
            gather_copy(h, hb).start()

    for h in range(DEPTH):
        gather_copy(h, h).start()

    def group(p, carry):
        for b in range(NBUF):
            visit(NBUF * p + b, b)
        return carry

    lax.fori_loop(0, NUM_CHUNKS // NBUF, group, 0)

    for b in range(NBUF):
        store_copy(NUM_CHUNKS - NBUF + b, b).wait()


@jax.jit
def _embed(x, table):
    mesh = plsc.VectorSubcoreMesh(
        core_axis_name="c", subcore_axis_name="s",
        num_cores=NUM_CORES, num_subcores=NUM_SUBCORES,
    )
    run = functools.partial(
        pl.kernel,
        out_type=jax.ShapeDtypeStruct((BATCH, HIST, EMBED_DIM), jnp.float32),
        mesh=mesh,
        scratch_types=[
            pltpu.VMEM((XROWS, HIST), jnp.int32),
            pltpu.VMEM((NBUF, HIST, EMBED_DIM), jnp.float32),
            [pltpu.SemaphoreType.DMA] * NBUF,
            [pltpu.SemaphoreType.DMA] * NBUF,
        ],
        compiler_params=pltpu.CompilerParams(use_tc_tiling_on_sc=False),
    )(_embed_kernel)
    return run(x, table)


def kernel(x, table):
    return _embed(x, table)


# speedup vs baseline: 1.2197x; 1.2197x over previous
import functools

import jax
import jax.numpy as jnp
from jax import lax
from jax.experimental import pallas as pl
from jax.experimental.pallas import tpu as pltpu
from jax.experimental.pallas import tpu_sc as plsc

VOCAB = 1000000
EMBED_DIM = 64
PADDED_DIM = 128
BATCH = 4096
HIST = 200

NUM_CORES = 2
NUM_SUBCORES = 16
NUM_WORKERS = NUM_CORES * NUM_SUBCORES

XROWS = BATCH // NUM_WORKERS
NUM_CHUNKS = XROWS
NBUF = 4
DEPTH = NBUF - 1


def _embed_kernel(x_hbm, table_hbm, out_hbm, idx_all, rows_v, gsems, ssems):
    wid = lax.axis_index("s") * NUM_CORES + lax.axis_index("c")
    wrow = wid * XROWS

    pltpu.sync_copy(x_hbm.at[pl.ds(wrow, XROWS)], idx_all)

    def gather_copy(c, b):
        src = table_hbm.at[idx_all.at[c]]
        return pltpu.make_async_copy(src, rows_v.at[b], gsems[b])

    def store_copy(c, b):
        dst = out_hbm.at[wrow + c]
        return pltpu.make_async_copy(rows_v.at[b], dst, ssems[b])

    def visit(c, b):
        gather_copy(c, b).wait()
        store_copy(c, b).start()
        h = c + DEPTH
        hb = (b + DEPTH) % NBUF

        @pl.when(h < NUM_CHUNKS)
        def _():
            @pl.when(h >= NBUF)
            def _():
                store_copy(h - NBUF, hb).wait()
            gather_copy(h, hb).start()

    for h in range(DEPTH):
        gather_copy(h, h).start()

    def group(p, carry):
        for b in range(NBUF):
            visit(NBUF * p + b, b)
        return carry

    lax.fori_loop(0, NUM_CHUNKS // NBUF, group, 0)

    for b in range(NBUF):
        store_copy(NUM_CHUNKS - NBUF + b, b).wait()


@jax.jit
def _embed(x, table):
    table_p = jnp.pad(table, ((0, 0), (0, PADDED_DIM - EMBED_DIM)))
    mesh = plsc.VectorSubcoreMesh(
        core_axis_name="c", subcore_axis_name="s",
        num_cores=NUM_CORES, num_subcores=NUM_SUBCORES,
    )
    run = functools.partial(
        pl.kernel,
        out_type=jax.ShapeDtypeStruct((BATCH, HIST, PADDED_DIM), jnp.float32),
        mesh=mesh,
        scratch_types=[
            pltpu.VMEM((XROWS, HIST), jnp.int32),
            pltpu.VMEM((NBUF, HIST, PADDED_DIM), jnp.float32),
            [pltpu.SemaphoreType.DMA] * NBUF,
            [pltpu.SemaphoreType.DMA] * NBUF,
        ],
        compiler_params=pltpu.CompilerParams(use_tc_tiling_on_sc=False),
    )(_embed_kernel)
    out_p = run(x, table_p)
    return out_p[:, :, :EMBED_DIM]


def kernel(x, table):
    return _embed(x, table)
